# Initial kernel scaffold; baseline (speedup 1.0000x reference)
import jax
import jax.numpy as jnp
from jax.experimental import pallas as pl


def kernel(x, W_sat, b_sat, W_nei, b_nei, W_fus, b_fus, W1, a_src1, a_dst1, bc1, g1, be1, W2, a_src2, a_dst2, bc2, g2, be2, Wf1, bf1, Wf2, bf2, edge_index):
    raise NotImplementedError("write your pallas kernel here")


# speedup vs baseline: 4.1223x; 4.1223x over previous
import functools

import jax
import jax.numpy as jnp
from jax import lax
from jax.experimental import pallas as pl
from jax.experimental.pallas import tpu as pltpu
from jax.experimental.pallas import tpu_sc as plsc

N = 10000
E = 320000
D_IN = 128
SAT = 64
H = 128
OUT = 64
NEG_SLOPE = 0.2
EPS_BN = 1e-5

E2 = E + N
NC = 2
NS = 16
B = 128
NB = 168
CE = NB * B
EPAD = NS * CE
NPAD = 10240
QR = NPAD // 4
ACCR = QR + 8
RPQ = QR // NS

ROWS_N = 10
BR = N // ROWS_N


def _tc1_body(x_ref, wsat_ref, bsat_ref, wnei_ref, bnei_ref, wfa_ref, wfb_ref,
              bfus_ref, w1_ref, asrc_ref, adst_ref,
              h0_ref, z_ref, zs_ref, zd_ref, gub_ref, mx_ref):
    i = pl.program_id(0)
    xb = x_ref[...]
    sat = jnp.maximum(
        jnp.dot(xb[:, :SAT], wsat_ref[...], preferred_element_type=jnp.float32)
        + bsat_ref[...], 0.0)
    nei = jnp.maximum(
        jnp.dot(xb[:, SAT:], wnei_ref[...], preferred_element_type=jnp.float32)
        + bnei_ref[...], 0.0)
    gl = (jnp.sum(sat * wfa_ref[...], axis=1, keepdims=True)
          + jnp.sum(nei * wfb_ref[...], axis=1, keepdims=True)
          + bfus_ref[0, 0])
    gate = jax.nn.sigmoid(gl)
    h = gate * sat + (1.0 - gate) * nei
    z = jnp.dot(h, w1_ref[...], preferred_element_type=jnp.float32)
    zs = jnp.sum(z * asrc_ref[...], axis=1, keepdims=True)
    zd = jnp.sum(z * adst_ref[...], axis=1, keepdims=True)
    h0_ref[...] = h
    z_ref[...] = z
    zs_ref[...] = zs
    zd_ref[...] = zd

    @pl.when(i == 0)
    def _():
        mx_ref[0] = -3e38
        mx_ref[1] = -3e38

    mx_ref[0] = jnp.maximum(mx_ref[0], jnp.max(zs))
    mx_ref[1] = jnp.maximum(mx_ref[1], jnp.max(zd))

    @pl.when(i == ROWS_N - 1)
    def _():
        gub_ref[...] = jnp.full((1, H), mx_ref[0] + mx_ref[1], jnp.float32)


def _tc1(x, wsat, bsat, wnei, bnei, wfa, wfb, bfus, w1, asrc, adst):
    full = lambda s: pl.BlockSpec(s, lambda i: (0,) * len(s))
    return pl.pallas_call(
        _tc1_body,
        grid=(ROWS_N,),
        in_specs=[
            pl.BlockSpec((BR, D_IN), lambda i: (i, 0)),
            full((SAT, H)), full((1, H)), full((D_IN - SAT, H)), full((1, H)),
            full((1, H)), full((1, H)), full((1, 1)), full((H, H)),
            full((1, H)), full((1, H)),
        ],
        out_specs=[
            pl.BlockSpec((BR, H), lambda i: (i, 0)),
            pl.BlockSpec((BR, H), lambda i: (i, 0)),
            pl.BlockSpec((BR, 1), lambda i: (i, 0)),
            pl.BlockSpec((BR, 1), lambda i: (i, 0)),
            pl.BlockSpec((1, H), lambda i: (0, 0)),
        ],
        out_shape=[
            jax.ShapeDtypeStruct((N, H), jnp.float32),
            jax.ShapeDtypeStruct((N, H), jnp.float32),
            jax.ShapeDtypeStruct((N, 1), jnp.float32),
            jax.ShapeDtypeStruct((N, 1), jnp.float32),
            jax.ShapeDtypeStruct((1, H), jnp.float32),
        ],
        scratch_shapes=[pltpu.SMEM((2,), jnp.float32)],
    )(x, wsat, bsat, wnei, bnei, wfa, wfb, bfus, w1, asrc, adst)


def _sc_body(z_h, zs_h, zd_h, g_h, src_h, dst_h, acc_o, wacc_o,
             src_v, dst_v, sidx_v, ldst_v, zs_v, zd_v, g_v, rows_v, srows_v,
             eerow_v, zv, acc, wacc, sem):
    cid = lax.axis_index("c")
    sid = lax.axis_index("s")

    pltpu.sync_copy(src_h.at[sid], src_v)
    pltpu.sync_copy(dst_h.at[sid], dst_v)
    pltpu.sync_copy(zs_h, zs_v)
    pltpu.sync_copy(zd_h, zd_v)
    pltpu.sync_copy(g_h.at[pl.ds(0, 16)], g_v)

    g16 = g_v[...]
    valid = E2 - sid * CE
    lanes = lax.iota(jnp.int32, 16)
    zero16 = jnp.zeros((16,), jnp.float32)
    base = sid * RPQ

    for p in range(2):
        q = 2 * p + cid
        lo = q * QR

        def zrow(r, c):
            for k in range(H // 16):
                srows_v[r, pl.ds(k * 16, 16)] = zero16
            return c

        lax.fori_loop(0, B, zrow, 0)

        def zw(r, c):
            zv[pl.ds(r * 16, 16)] = zero16
            return c

        lax.fori_loop(0, RPQ // 16, zw, 0)
        pltpu.sync_copy(srows_v, acc.at[pl.ds(base, B)])
        pltpu.sync_copy(srows_v.at[pl.ds(0, RPQ - B)],
                        acc.at[pl.ds(base + B, RPQ - B)])
        pltpu.sync_copy(zv, wacc.at[pl.ds(base, RPQ)])
        plsc.subcore_barrier()

        @pl.loop(0, NB)
        def batch(j):
            pltpu.sync_copy(src_h.at[sid, j], sidx_v)
            cp = pltpu.async_copy(z_h.at[sidx_v], rows_v, sem)

            def group_a(kk, c2):
                off = kk * 16
                s16 = src_v[j, pl.ds(off, 16)]
                d16 = dst_v[j, pl.ds(off, 16)]
                zsg = plsc.load_gather(zs_v, [s16])
                zdg = plsc.load_gather(zd_v, [d16])
                e = zsg + zdg
                e = jnp.where(e >= 0.0, e, e * NEG_SLOPE)
                ee = jnp.exp(e - g16)
                ee = jnp.where((j * B + off + lanes) < valid, ee, 0.0)
                ld = d16 - lo
                ld = jnp.where((ld >= 0) & (ld < QR), ld, QR)
                ldst_v[pl.ds(off, 16)] = ld
                eerow_v[pl.ds(off, 16)] = ee
                return c2

            lax.fori_loop(0, B // 16, group_a, 0)
            cp.wait()

            def group_b(kk, c2):
                off = kk * 16
                ee = eerow_v[pl.ds(off, 16)]
                for r in range(16):
                    a = ee[r]
                    row = off + r
                    for k in range(H // 16):
                        srows_v[row, pl.ds(k * 16, 16)] = (
                            rows_v[row, pl.ds(k * 16, 16)] * a)
                return c2

            lax.fori_loop(0, B // 16, group_b, 0)
            pltpu.sync_copy(srows_v, acc.at[ldst_v], add=True)
            pltpu.sync_copy(eerow_v, wacc.at[ldst_v], add=True)

        plsc.subcore_barrier()
        pltpu.sync_copy(acc.at[pl.ds(base, RPQ)],
                        acc_o.at[pl.ds(lo + base, RPQ)])
        pltpu.sync_copy(wacc.at[pl.ds(base, RPQ)], zv)
        pltpu.sync_copy(zv, wacc_o.at[pl.ds(lo + base, RPQ)])
        plsc.subcore_barrier()


def _make_sc_edges():
    mesh = plsc.VectorSubcoreMesh(core_axis_name="c", subcore_axis_name="s")
    kfn = functools.partial(
        pl.kernel,
        out_type=[
            jax.ShapeDtypeStruct((NPAD, H), jnp.float32),
            jax.ShapeDtypeStruct((NPAD,), jnp.float32),
        ],
        mesh=mesh,
        compiler_params=pltpu.CompilerParams(needs_layout_passes=False,
                                             use_tc_tiling_on_sc=False),
        scratch_types=[
            pltpu.VMEM((NB, B), jnp.int32),
            pltpu.VMEM((NB, B), jnp.int32),
            pltpu.VMEM((B,), jnp.int32),
            pltpu.VMEM((B,), jnp.int32),
            pltpu.VMEM((N,), jnp.float32),
            pltpu.VMEM((N,), jnp.float32),
            pltpu.VMEM((16,), jnp.float32),
            pltpu.VMEM((B, H), jnp.float32),
            pltpu.VMEM((B, H), jnp.float32),
            pltpu.VMEM((B,), jnp.float32),
            pltpu.VMEM((RPQ,), jnp.float32),
            pltpu.VMEM_SHARED((ACCR, H), jnp.float32),
            pltpu.VMEM_SHARED((ACCR,), jnp.float32),
            pltpu.SemaphoreType.DMA,
        ],
    )(_sc_body)
    return kfn


_sc_edges = _make_sc_edges()


def _tc2a_body(a_ref, w_ref, bc_ref, y_ref, st_ref, s1_ref, s2_ref):
    i = pl.program_id(0)
    y = a_ref[...] / (w_ref[...] + 1e-16) + bc_ref[...]
    y_ref[...] = y

    @pl.when(i == 0)
    def _():
        s1_ref[...] = jnp.zeros((8, H), jnp.float32)
        s2_ref[...] = jnp.zeros((8, H), jnp.float32)

    s1_ref[0:1, :] = s1_ref[0:1, :] + jnp.sum(y, axis=0, keepdims=True)
    s2_ref[0:1, :] = s2_ref[0:1, :] + jnp.sum(y * y, axis=0, keepdims=True)

    @pl.when(i == ROWS_N - 1)
    def _():
        st_ref[...] = jnp.concatenate([s1_ref[0:1, :], s2_ref[0:1, :]], axis=0)


def _tc2a(accs, waccs, bc):
    return pl.pallas_call(
        _tc2a_body,
        grid=(ROWS_N,),
        in_specs=[
            pl.BlockSpec((BR, H), lambda i: (i, 0)),
            pl.BlockSpec((BR, 1), lambda i: (i, 0)),
            pl.BlockSpec((1, H), lambda i: (0, 0)),
        ],
        out_specs=[
            pl.BlockSpec((BR, H), lambda i: (i, 0)),
            pl.BlockSpec((2, H), lambda i: (0, 0)),
        ],
        out_shape=[
            jax.ShapeDtypeStruct((N, H), jnp.float32),
            jax.ShapeDtypeStruct((2, H), jnp.float32),
        ],
        scratch_shapes=[
            pltpu.VMEM((8, H), jnp.float32),
            pltpu.VMEM((8, H), jnp.float32),
        ],
    )(accs, waccs, bc)


def _tc2b_body(y_ref, st_ref, res_ref, g_ref, be_ref, w2_ref, asrc_ref,
               adst_ref, h1_ref, z_ref, zs_ref, zd_ref, gub_ref, mx_ref):
    i = pl.program_id(0)
    mu = st_ref[0:1, :] * (1.0 / N)
    var = st_ref[1:2, :] * (1.0 / N) - mu * mu
    rs = lax.rsqrt(var + EPS_BN)
    y = y_ref[...]
    h1 = jnp.maximum(g_ref[...] * (y - mu) * rs + be_ref[...], 0.0) + res_ref[...]
    z = jnp.dot(h1, w2_ref[...], preferred_element_type=jnp.float32)
    zs = jnp.sum(z * asrc_ref[...], axis=1, keepdims=True)
    zd = jnp.sum(z * adst_ref[...], axis=1, keepdims=True)
    h1_ref[...] = h1
    z_ref[...] = z
    zs_ref[...] = zs
    zd_ref[...] = zd

    @pl.when(i == 0)
    def _():
        mx_ref[0] = -3e38
        mx_ref[1] = -3e38

    mx_ref[0] = jnp.maximum(mx_ref[0], jnp.max(zs))
    mx_ref[1] = jnp.maximum(mx_ref[1], jnp.max(zd))

    @pl.when(i == ROWS_N - 1)
    def _():
        gub_ref[...] = jnp.full((1, H), mx_ref[0] + mx_ref[1], jnp.float32)


def _tc2b(y, st, res, g, be, w2, asrc, adst):
    full = lambda s: pl.BlockSpec(s, lambda i: (0,) * len(s))
    return pl.pallas_call(
        _tc2b_body,
        grid=(ROWS_N,),
        in_specs=[
            pl.BlockSpec((BR, H), lambda i: (i, 0)),
            full((2, H)),
            pl.BlockSpec((BR, H), lambda i: (i, 0)),
            full((1, H)), full((1, H)), full((H, H)), full((1, H)), full((1, H)),
        ],
        out_specs=[
            pl.BlockSpec((BR, H), lambda i: (i, 0)),
            pl.BlockSpec((BR, H), lambda i: (i, 0)),
            pl.BlockSpec((BR, 1), lambda i: (i, 0)),
            pl.BlockSpec((BR, 1), lambda i: (i, 0)),
            pl.BlockSpec((1, H), lambda i: (0, 0)),
        ],
        out_shape=[
            jax.ShapeDtypeStruct((N, H), jnp.float32),
            jax.ShapeDtypeStruct((N, H), jnp.float32),
            jax.ShapeDtypeStruct((N, 1), jnp.float32),
            jax.ShapeDtypeStruct((N, 1), jnp.float32),
            jax.ShapeDtypeStruct((1, H), jnp.float32),
        ],
        scratch_shapes=[pltpu.SMEM((2,), jnp.float32)],
    )(y, st, res, g, be, w2, asrc, adst)


def _tc3_body(y_ref, st_ref, res_ref, g_ref, be_ref, wf1_ref, bf1_ref,
              wf2_ref, bf2_ref, out_ref):
    mu = st_ref[0:1, :] * (1.0 / N)
    var = st_ref[1:2, :] * (1.0 / N) - mu * mu
    rs = lax.rsqrt(var + EPS_BN)
    y = y_ref[...]
    h2 = jnp.maximum(g_ref[...] * (y - mu) * rs + be_ref[...], 0.0) + res_ref[...]
    f = jnp.maximum(
        jnp.dot(h2, wf1_ref[...], preferred_element_type=jnp.float32)
        + bf1_ref[...], 0.0)
    out_ref[...] = (jnp.dot(f, wf2_ref[...], preferred_element_type=jnp.float32)
                    + bf2_ref[...])


def _tc3(y, st, res, g, be, wf1, bf1, wf2, bf2):
    full = lambda s: pl.BlockSpec(s, lambda i: (0,) * len(s))
    return pl.pallas_call(
        _tc3_body,
        grid=(ROWS_N,),
        in_specs=[
            pl.BlockSpec((BR, H), lambda i: (i, 0)),
            full((2, H)),
            pl.BlockSpec((BR, H), lambda i: (i, 0)),
            full((1, H)), full((1, H)), full((H, H)), full((1, H)),
            full((H, OUT)), full((1, OUT)),
        ],
        out_specs=pl.BlockSpec((BR, OUT), lambda i: (i, 0)),
        out_shape=jax.ShapeDtypeStruct((N, OUT), jnp.float32),
    )(y, st, res, g, be, wf1, bf1, wf2, bf2)


def kernel(x, W_sat, b_sat, W_nei, b_nei, W_fus, b_fus, W1, a_src1, a_dst1,
           bc1, g1, be1, W2, a_src2, a_dst2, bc2, g2, be2, Wf1, bf1, Wf2, bf2,
           edge_index):
    row = lambda v: v.reshape(1, -1)
    si = jnp.arange(N, dtype=edge_index.dtype)
    pad = jnp.zeros((EPAD - E2,), edge_index.dtype)
    srcb = jnp.concatenate([edge_index[0], si, pad]).reshape(NS, NB, B)
    dstb = jnp.concatenate([edge_index[1], si, pad]).reshape(NS, NB, B)

    h0, z1, zs1, zd1, gub1 = _tc1(
        x, W_sat, row(b_sat), W_nei, row(b_nei),
        row(W_fus[:H, 0]), row(W_fus[H:, 0]), b_fus.reshape(1, 1),
        W1, row(a_src1), row(a_dst1))
    acc1, wacc1 = _sc_edges(z1, zs1.reshape(N), zd1.reshape(N),
                            gub1.reshape(H), srcb, dstb)
    y1, st1 = _tc2a(acc1[:N], wacc1[:N].reshape(N, 1), row(bc1))
    h1, z2, zs2, zd2, gub2 = _tc2b(y1, st1, h0, row(g1), row(be1),
                                   W2, row(a_src2), row(a_dst2))
    acc2, wacc2 = _sc_edges(z2, zs2.reshape(N), zd2.reshape(N),
                            gub2.reshape(H), srcb, dstb)
    y2, st2 = _tc2a(acc2[:N], wacc2[:N].reshape(N, 1), row(bc2))
    return _tc3(y2, st2, h1, row(g2), row(be2), Wf1, row(bf1), Wf2, row(bf2))


# speedup vs baseline: 4.1378x; 1.0038x over previous
import functools

import jax
import jax.numpy as jnp
from jax import lax
from jax.experimental import pallas as pl
from jax.experimental.pallas import tpu as pltpu
from jax.experimental.pallas import tpu_sc as plsc

N = 10000
E = 320000
D_IN = 128
SAT = 64
H = 128
OUT = 64
NEG_SLOPE = 0.2
EPS_BN = 1e-5

E2 = E + N
NC = 2
NS = 16
B = 128
NB = 168
CE = NB * B
EPAD = NS * CE
NPAD = 10240
QR = NPAD // 4
ACCR = QR
RPQ = QR // NS

ROWS_N = 10
BR = N // ROWS_N


def _tc1_body(x_ref, wsat_ref, bsat_ref, wnei_ref, bnei_ref, wfa_ref, wfb_ref,
              bfus_ref, w1_ref, asrc_ref, adst_ref,
              h0_ref, z_ref, zs_ref, zd_ref, gub_ref, mx_ref):
    i = pl.program_id(0)
    xb = x_ref[...]
    sat = jnp.maximum(
        jnp.dot(xb[:, :SAT], wsat_ref[...], preferred_element_type=jnp.float32)
        + bsat_ref[...], 0.0)
    nei = jnp.maximum(
        jnp.dot(xb[:, SAT:], wnei_ref[...], preferred_element_type=jnp.float32)
        + bnei_ref[...], 0.0)
    gl = (jnp.sum(sat * wfa_ref[...], axis=1, keepdims=True)
          + jnp.sum(nei * wfb_ref[...], axis=1, keepdims=True)
          + bfus_ref[0, 0])
    gate = jax.nn.sigmoid(gl)
    h = gate * sat + (1.0 - gate) * nei
    z = jnp.dot(h, w1_ref[...], preferred_element_type=jnp.float32)
    zs = jnp.sum(z * asrc_ref[...], axis=1, keepdims=True)
    zd = jnp.sum(z * adst_ref[...], axis=1, keepdims=True)
    h0_ref[...] = h
    z_ref[...] = z
    zs_ref[...] = zs
    zd_ref[...] = zd

    @pl.when(i == 0)
    def _():
        mx_ref[0] = -3e38
        mx_ref[1] = -3e38

    mx_ref[0] = jnp.maximum(mx_ref[0], jnp.max(zs))
    mx_ref[1] = jnp.maximum(mx_ref[1], jnp.max(zd))

    @pl.when(i == ROWS_N - 1)
    def _():
        gub_ref[...] = jnp.full((1, H), mx_ref[0] + mx_ref[1], jnp.float32)


def _tc1(x, wsat, bsat, wnei, bnei, wfa, wfb, bfus, w1, asrc, adst):
    full = lambda s: pl.BlockSpec(s, lambda i: (0,) * len(s))
    return pl.pallas_call(
        _tc1_body,
        grid=(ROWS_N,),
        in_specs=[
            pl.BlockSpec((BR, D_IN), lambda i: (i, 0)),
            full((SAT, H)), full((1, H)), full((D_IN - SAT, H)), full((1, H)),
            full((1, H)), full((1, H)), full((1, 1)), full((H, H)),
            full((1, H)), full((1, H)),
        ],
        out_specs=[
            pl.BlockSpec((BR, H), lambda i: (i, 0)),
            pl.BlockSpec((BR, H), lambda i: (i, 0)),
            pl.BlockSpec((BR, 1), lambda i: (i, 0)),
            pl.BlockSpec((BR, 1), lambda i: (i, 0)),
            pl.BlockSpec((1, H), lambda i: (0, 0)),
        ],
        out_shape=[
            jax.ShapeDtypeStruct((N, H), jnp.float32),
            jax.ShapeDtypeStruct((N, H), jnp.float32),
            jax.ShapeDtypeStruct((N, 1), jnp.float32),
            jax.ShapeDtypeStruct((N, 1), jnp.float32),
            jax.ShapeDtypeStruct((1, H), jnp.float32),
        ],
        scratch_shapes=[pltpu.SMEM((2,), jnp.float32)],
    )(x, wsat, bsat, wnei, bnei, wfa, wfb, bfus, w1, asrc, adst)


def _sc_body(z_h, zs_h, zd_h, g_h, src_h, dst_h, acc_o, wacc_o,
             src_v, dst_v, sidx_v, ldst_v, zs_v, zd_v, g_v, rows_v, srows_v,
             eerow_v, zv, acc, wacc, sem):
    cid = lax.axis_index("c")
    sid = lax.axis_index("s")

    pltpu.sync_copy(src_h.at[sid], src_v)
    pltpu.sync_copy(dst_h.at[sid], dst_v)
    pltpu.sync_copy(zs_h, zs_v)
    pltpu.sync_copy(zd_h, zd_v)
    pltpu.sync_copy(g_h.at[pl.ds(0, 16)], g_v)

    g16 = g_v[...]
    valid = E2 - sid * CE
    lanes = lax.iota(jnp.int32, 16)
    zero16 = jnp.zeros((16,), jnp.float32)
    base = sid * RPQ

    for p in range(2):
        q = 2 * p + cid
        lo = q * QR

        def zrow(r, c):
            for k in range(H // 16):
                srows_v[r, pl.ds(k * 16, 16)] = zero16
            return c

        lax.fori_loop(0, B, zrow, 0)

        def zw(r, c):
            zv[pl.ds(r * 16, 16)] = zero16
            return c

        lax.fori_loop(0, RPQ // 16, zw, 0)
        pltpu.sync_copy(srows_v, acc.at[pl.ds(base, B)])
        pltpu.sync_copy(srows_v.at[pl.ds(0, RPQ - B)],
                        acc.at[pl.ds(base + B, RPQ - B)])
        pltpu.sync_copy(zv, wacc.at[pl.ds(base, RPQ)])
        plsc.subcore_barrier()

        @pl.loop(0, NB)
        def batch(j):
            pltpu.sync_copy(src_h.at[sid, j], sidx_v)
            cp = pltpu.async_copy(z_h.at[sidx_v], rows_v, sem)

            def group_a(kk, c2):
                off = kk * 16
                s16 = src_v[j, pl.ds(off, 16)]
                d16 = dst_v[j, pl.ds(off, 16)]
                zsg = plsc.load_gather(zs_v, [s16])
                zdg = plsc.load_gather(zd_v, [d16])
                e = zsg + zdg
                e = jnp.where(e >= 0.0, e, e * NEG_SLOPE)
                ee = jnp.exp(e - g16)
                ee = jnp.where((j * B + off + lanes) < valid, ee, 0.0)
                ld = d16 - lo
                ld = jnp.where((ld >= 0) & (ld < QR), ld, -1)
                ldst_v[pl.ds(off, 16)] = ld
                eerow_v[pl.ds(off, 16)] = ee
                return c2

            lax.fori_loop(0, B // 16, group_a, 0)
            cp.wait()

            def group_b(kk, c2):
                off = kk * 16
                ee = eerow_v[pl.ds(off, 16)]
                for r in range(16):
                    a = ee[r]
                    row = off + r
                    for k in range(H // 16):
                        srows_v[row, pl.ds(k * 16, 16)] = (
                            rows_v[row, pl.ds(k * 16, 16)] * a)
                return c2

            lax.fori_loop(0, B // 16, group_b, 0)
            fidx = plsc.Indices(ldst_v, ignored_value=-1)
            pltpu.sync_copy(srows_v, acc.at[fidx], add=True)
            pltpu.sync_copy(eerow_v, wacc.at[fidx], add=True)

        plsc.subcore_barrier()
        pltpu.sync_copy(acc.at[pl.ds(base, RPQ)],
                        acc_o.at[pl.ds(lo + base, RPQ)])
        pltpu.sync_copy(wacc.at[pl.ds(base, RPQ)], zv)
        pltpu.sync_copy(zv, wacc_o.at[pl.ds(lo + base, RPQ)])
        plsc.subcore_barrier()


def _make_sc_edges():
    mesh = plsc.VectorSubcoreMesh(core_axis_name="c", subcore_axis_name="s")
    kfn = functools.partial(
        pl.kernel,
        out_type=[
            jax.ShapeDtypeStruct((NPAD, H), jnp.float32),
            jax.ShapeDtypeStruct((NPAD,), jnp.float32),
        ],
        mesh=mesh,
        compiler_params=pltpu.CompilerParams(needs_layout_passes=False,
                                             use_tc_tiling_on_sc=False),
        scratch_types=[
            pltpu.VMEM((NB, B), jnp.int32),
            pltpu.VMEM((NB, B), jnp.int32),
            pltpu.VMEM((B,), jnp.int32),
            pltpu.VMEM((B,), jnp.int32),
            pltpu.VMEM((N,), jnp.float32),
            pltpu.VMEM((N,), jnp.float32),
            pltpu.VMEM((16,), jnp.float32),
            pltpu.VMEM((B, H), jnp.float32),
            pltpu.VMEM((B, H), jnp.float32),
            pltpu.VMEM((B,), jnp.float32),
            pltpu.VMEM((RPQ,), jnp.float32),
            pltpu.VMEM_SHARED((ACCR, H), jnp.float32),
            pltpu.VMEM_SHARED((ACCR,), jnp.float32),
            pltpu.SemaphoreType.DMA,
        ],
    )(_sc_body)
    return kfn


_sc_edges = _make_sc_edges()


def _tc2a_body(a_ref, w_ref, bc_ref, y_ref, st_ref, s1_ref, s2_ref):
    i = pl.program_id(0)
    y = a_ref[...] / (w_ref[...] + 1e-16) + bc_ref[...]
    y_ref[...] = y

    @pl.when(i == 0)
    def _():
        s1_ref[...] = jnp.zeros((8, H), jnp.float32)
        s2_ref[...] = jnp.zeros((8, H), jnp.float32)

    s1_ref[0:1, :] = s1_ref[0:1, :] + jnp.sum(y, axis=0, keepdims=True)
    s2_ref[0:1, :] = s2_ref[0:1, :] + jnp.sum(y * y, axis=0, keepdims=True)

    @pl.when(i == ROWS_N - 1)
    def _():
        st_ref[...] = jnp.concatenate([s1_ref[0:1, :], s2_ref[0:1, :]], axis=0)


def _tc2a(accs, waccs, bc):
    return pl.pallas_call(
        _tc2a_body,
        grid=(ROWS_N,),
        in_specs=[
            pl.BlockSpec((BR, H), lambda i: (i, 0)),
            pl.BlockSpec((BR, 1), lambda i: (i, 0)),
            pl.BlockSpec((1, H), lambda i: (0, 0)),
        ],
        out_specs=[
            pl.BlockSpec((BR, H), lambda i: (i, 0)),
            pl.BlockSpec((2, H), lambda i: (0, 0)),
        ],
        out_shape=[
            jax.ShapeDtypeStruct((N, H), jnp.float32),
            jax.ShapeDtypeStruct((2, H), jnp.float32),
        ],
        scratch_shapes=[
            pltpu.VMEM((8, H), jnp.float32),
            pltpu.VMEM((8, H), jnp.float32),
        ],
    )(accs, waccs, bc)


def _tc2b_body(y_ref, st_ref, res_ref, g_ref, be_ref, w2_ref, asrc_ref,
               adst_ref, h1_ref, z_ref, zs_ref, zd_ref, gub_ref, mx_ref):
    i = pl.program_id(0)
    mu = st_ref[0:1, :] * (1.0 / N)
    var = st_ref[1:2, :] * (1.0 / N) - mu * mu
    rs = lax.rsqrt(var + EPS_BN)
    y = y_ref[...]
    h1 = jnp.maximum(g_ref[...] * (y - mu) * rs + be_ref[...], 0.0) + res_ref[...]
    z = jnp.dot(h1, w2_ref[...], preferred_element_type=jnp.float32)
    zs = jnp.sum(z * asrc_ref[...], axis=1, keepdims=True)
    zd = jnp.sum(z * adst_ref[...], axis=1, keepdims=True)
    h1_ref[...] = h1
    z_ref[...] = z
    zs_ref[...] = zs
    zd_ref[...] = zd

    @pl.when(i == 0)
    def _():
        mx_ref[0] = -3e38
        mx_ref[1] = -3e38

    mx_ref[0] = jnp.maximum(mx_ref[0], jnp.max(zs))
    mx_ref[1] = jnp.maximum(mx_ref[1], jnp.max(zd))

    @pl.when(i == ROWS_N - 1)
    def _():
        gub_ref[...] = jnp.full((1, H), mx_ref[0] + mx_ref[1], jnp.float32)


def _tc2b(y, st, res, g, be, w2, asrc, adst):
    full = lambda s: pl.BlockSpec(s, lambda i: (0,) * len(s))
    return pl.pallas_call(
        _tc2b_body,
        grid=(ROWS_N,),
        in_specs=[
            pl.BlockSpec((BR, H), lambda i: (i, 0)),
            full((2, H)),
            pl.BlockSpec((BR, H), lambda i: (i, 0)),
            full((1, H)), full((1, H)), full((H, H)), full((1, H)), full((1, H)),
        ],
        out_specs=[
            pl.BlockSpec((BR, H), lambda i: (i, 0)),
            pl.BlockSpec((BR, H), lambda i: (i, 0)),
            pl.BlockSpec((BR, 1), lambda i: (i, 0)),
            pl.BlockSpec((BR, 1), lambda i: (i, 0)),
            pl.BlockSpec((1, H), lambda i: (0, 0)),
        ],
        out_shape=[
            jax.ShapeDtypeStruct((N, H), jnp.float32),
            jax.ShapeDtypeStruct((N, H), jnp.float32),
            jax.ShapeDtypeStruct((N, 1), jnp.float32),
            jax.ShapeDtypeStruct((N, 1), jnp.float32),
            jax.ShapeDtypeStruct((1, H), jnp.float32),
        ],
        scratch_shapes=[pltpu.SMEM((2,), jnp.float32)],
    )(y, st, res, g, be, w2, asrc, adst)


def _tc3_body(y_ref, st_ref, res_ref, g_ref, be_ref, wf1_ref, bf1_ref,
              wf2_ref, bf2_ref, out_ref):
    mu = st_ref[0:1, :] * (1.0 / N)
    var = st_ref[1:2, :] * (1.0 / N) - mu * mu
    rs = lax.rsqrt(var + EPS_BN)
    y = y_ref[...]
    h2 = jnp.maximum(g_ref[...] * (y - mu) * rs + be_ref[...], 0.0) + res_ref[...]
    f = jnp.maximum(
        jnp.dot(h2, wf1_ref[...], preferred_element_type=jnp.float32)
        + bf1_ref[...], 0.0)
    out_ref[...] = (jnp.dot(f, wf2_ref[...], preferred_element_type=jnp.float32)
                    + bf2_ref[...])


def _tc3(y, st, res, g, be, wf1, bf1, wf2, bf2):
    full = lambda s: pl.BlockSpec(s, lambda i: (0,) * len(s))
    return pl.pallas_call(
        _tc3_body,
        grid=(ROWS_N,),
        in_specs=[
            pl.BlockSpec((BR, H), lambda i: (i, 0)),
            full((2, H)),
            pl.BlockSpec((BR, H), lambda i: (i, 0)),
            full((1, H)), full((1, H)), full((H, H)), full((1, H)),
            full((H, OUT)), full((1, OUT)),
        ],
        out_specs=pl.BlockSpec((BR, OUT), lambda i: (i, 0)),
        out_shape=jax.ShapeDtypeStruct((N, OUT), jnp.float32),
    )(y, st, res, g, be, wf1, bf1, wf2, bf2)


def kernel(x, W_sat, b_sat, W_nei, b_nei, W_fus, b_fus, W1, a_src1, a_dst1,
           bc1, g1, be1, W2, a_src2, a_dst2, bc2, g2, be2, Wf1, bf1, Wf2, bf2,
           edge_index):
    row = lambda v: v.reshape(1, -1)
    si = jnp.arange(N, dtype=edge_index.dtype)
    pad = jnp.zeros((EPAD - E2,), edge_index.dtype)
    srcb = jnp.concatenate([edge_index[0], si, pad]).reshape(NS, NB, B)
    dstb = jnp.concatenate([edge_index[1], si, pad]).reshape(NS, NB, B)

    h0, z1, zs1, zd1, gub1 = _tc1(
        x, W_sat, row(b_sat), W_nei, row(b_nei),
        row(W_fus[:H, 0]), row(W_fus[H:, 0]), b_fus.reshape(1, 1),
        W1, row(a_src1), row(a_dst1))
    acc1, wacc1 = _sc_edges(z1, zs1.reshape(N), zd1.reshape(N),
                            gub1.reshape(H), srcb, dstb)
    y1, st1 = _tc2a(acc1[:N], wacc1[:N].reshape(N, 1), row(bc1))
    h1, z2, zs2, zd2, gub2 = _tc2b(y1, st1, h0, row(g1), row(be1),
                                   W2, row(a_src2), row(a_dst2))
    acc2, wacc2 = _sc_edges(z2, zs2.reshape(N), zd2.reshape(N),
                            gub2.reshape(H), srcb, dstb)
    y2, st2 = _tc2a(acc2[:N], wacc2[:N].reshape(N, 1), row(bc2))
    return _tc3(y2, st2, h1, row(g2), row(be2), Wf1, row(bf1), Wf2, row(bf2))


# speedup vs baseline: 4.3947x; 1.0621x over previous
import functools

import jax
import jax.numpy as jnp
from jax import lax
from jax.experimental import pallas as pl
from jax.experimental.pallas import tpu as pltpu
from jax.experimental.pallas import tpu_sc as plsc

N = 10000
E = 320000
D_IN = 128
SAT = 64
H = 128
OUT = 64
NEG_SLOPE = 0.2
EPS_BN = 1e-5

E2 = E + N
NC = 2
NS = 16
B = 128
NB = 168
CE = NB * B
EPAD = NS * CE
NPAD = 10240
QR = NPAD // 4
ACCR = QR
RPQ = QR // NS

ROWS_N = 10
BR = N // ROWS_N


def _tc1_body(x_ref, wsat_ref, bsat_ref, wnei_ref, bnei_ref, wfa_ref, wfb_ref,
              bfus_ref, w1_ref, asrc_ref, adst_ref,
              h0_ref, z_ref, zs_ref, zd_ref, gub_ref, mx_ref):
    i = pl.program_id(0)
    xb = x_ref[...]
    sat = jnp.maximum(
        jnp.dot(xb[:, :SAT], wsat_ref[...], preferred_element_type=jnp.float32)
        + bsat_ref[...], 0.0)
    nei = jnp.maximum(
        jnp.dot(xb[:, SAT:], wnei_ref[...], preferred_element_type=jnp.float32)
        + bnei_ref[...], 0.0)
    gl = (jnp.sum(sat * wfa_ref[...], axis=1, keepdims=True)
          + jnp.sum(nei * wfb_ref[...], axis=1, keepdims=True)
          + bfus_ref[0, 0])
    gate = jax.nn.sigmoid(gl)
    h = gate * sat + (1.0 - gate) * nei
    z = jnp.dot(h, w1_ref[...], preferred_element_type=jnp.float32)
    zs = jnp.sum(z * asrc_ref[...], axis=1, keepdims=True)
    zd = jnp.sum(z * adst_ref[...], axis=1, keepdims=True)
    h0_ref[...] = h
    z_ref[...] = z
    zs_ref[...] = zs
    zd_ref[...] = zd

    @pl.when(i == 0)
    def _():
        mx_ref[0] = -3e38
        mx_ref[1] = -3e38

    mx_ref[0] = jnp.maximum(mx_ref[0], jnp.max(zs))
    mx_ref[1] = jnp.maximum(mx_ref[1], jnp.max(zd))

    @pl.when(i == ROWS_N - 1)
    def _():
        gub_ref[...] = jnp.full((1, H), mx_ref[0] + mx_ref[1], jnp.float32)


def _tc1(x, wsat, bsat, wnei, bnei, wfa, wfb, bfus, w1, asrc, adst):
    full = lambda s: pl.BlockSpec(s, lambda i: (0,) * len(s))
    return pl.pallas_call(
        _tc1_body,
        grid=(ROWS_N,),
        in_specs=[
            pl.BlockSpec((BR, D_IN), lambda i: (i, 0)),
            full((SAT, H)), full((1, H)), full((D_IN - SAT, H)), full((1, H)),
            full((1, H)), full((1, H)), full((1, 1)), full((H, H)),
            full((1, H)), full((1, H)),
        ],
        out_specs=[
            pl.BlockSpec((BR, H), lambda i: (i, 0)),
            pl.BlockSpec((BR, H), lambda i: (i, 0)),
            pl.BlockSpec((BR, 1), lambda i: (i, 0)),
            pl.BlockSpec((BR, 1), lambda i: (i, 0)),
            pl.BlockSpec((1, H), lambda i: (0, 0)),
        ],
        out_shape=[
            jax.ShapeDtypeStruct((N, H), jnp.float32),
            jax.ShapeDtypeStruct((N, H), jnp.float32),
            jax.ShapeDtypeStruct((N, 1), jnp.float32),
            jax.ShapeDtypeStruct((N, 1), jnp.float32),
            jax.ShapeDtypeStruct((1, H), jnp.float32),
        ],
        scratch_shapes=[pltpu.SMEM((2,), jnp.float32)],
    )(x, wsat, bsat, wnei, bnei, wfa, wfb, bfus, w1, asrc, adst)


def _sc_body(z_h, zs_h, zd_h, g_h, src_h, dst_h, acc_o, wacc_o,
             sidx0, sidx1, didx0, didx1, ldst0, ldst1, zs_v, zd_v, g_v,
             rows0, rows1, srows0, srows1, ee0, ee1, zv, acc, wacc,
             gsem0, gsem1, ssem0, ssem1):
    cid = lax.axis_index("c")
    sid = lax.axis_index("s")

    pltpu.sync_copy(zs_h, zs_v)
    pltpu.sync_copy(zd_h, zd_v)
    pltpu.sync_copy(g_h.at[pl.ds(0, 16)], g_v)

    g16 = g_v[...]
    valid = E2 - sid * CE
    lanes = lax.iota(jnp.int32, 16)
    zero16 = jnp.zeros((16,), jnp.float32)
    base = sid * RPQ
    bufs = ((sidx0, didx0, rows0, srows0, ldst0, ee0, gsem0, ssem0),
            (sidx1, didx1, rows1, srows1, ldst1, ee1, gsem1, ssem1))

    for p in range(2):
        q = 2 * p + cid
        lo = q * QR

        def zrow(r, c):
            for k in range(H // 16):
                srows0[r, pl.ds(k * 16, 16)] = zero16
            return c

        lax.fori_loop(0, B, zrow, 0)

        def zw(r, c):
            zv[pl.ds(r * 16, 16)] = zero16
            return c

        lax.fori_loop(0, RPQ // 16, zw, 0)
        pltpu.sync_copy(srows0, acc.at[pl.ds(base, B)])
        pltpu.sync_copy(srows0.at[pl.ds(0, RPQ - B)],
                        acc.at[pl.ds(base + B, RPQ - B)])
        pltpu.sync_copy(zv, wacc.at[pl.ds(base, RPQ)])
        plsc.subcore_barrier()

        pltpu.sync_copy(src_h.at[sid, 0], sidx0)
        pltpu.sync_copy(dst_h.at[sid, 0], didx0)
        pltpu.async_copy(z_h.at[sidx0], rows0, gsem0)

        @pl.loop(0, NB // 2)
        def pair(jj):
            for b in range(2):
                (sidx_b, didx_b, rows_b, srows_b, ldst_b, ee_b, gsem_b,
                 ssem_b) = bufs[b]
                sidx_n, didx_n, rows_n = (bufs[1 - b][0], bufs[1 - b][1],
                                          bufs[1 - b][2])
                gsem_n = bufs[1 - b][6]
                j = 2 * jj + b
                fidx = plsc.Indices(ldst_b, ignored_value=-1)

                @pl.when(j + 1 < NB)
                def _():
                    pltpu.sync_copy(src_h.at[sid, j + 1], sidx_n)
                    pltpu.sync_copy(dst_h.at[sid, j + 1], didx_n)
                    pltpu.async_copy(z_h.at[sidx_n], rows_n, gsem_n)

                @pl.when(j >= 2)
                def _():
                    pltpu.make_async_copy(
                        srows_b, acc.at[fidx], ssem_b).wait()
                    pltpu.make_async_copy(
                        ee_b, wacc.at[fidx], ssem_b).wait()

                def group_a(kk, c2):
                    off = kk * 16
                    s16 = sidx_b[pl.ds(off, 16)]
                    d16 = didx_b[pl.ds(off, 16)]
                    zsg = plsc.load_gather(zs_v, [s16])
                    zdg = plsc.load_gather(zd_v, [d16])
                    e = zsg + zdg
                    e = jnp.where(e >= 0.0, e, e * NEG_SLOPE)
                    ee = jnp.exp(e - g16)
                    ee = jnp.where((j * B + off + lanes) < valid, ee, 0.0)
                    ld = d16 - lo
                    ld = jnp.where((ld >= 0) & (ld < QR), ld, -1)
                    ldst_b[pl.ds(off, 16)] = ld
                    ee_b[pl.ds(off, 16)] = ee
                    return c2

                lax.fori_loop(0, B // 16, group_a, 0)
                pltpu.make_async_copy(z_h.at[sidx_b], rows_b, gsem_b).wait()

                def group_b(kk, c2):
                    off = kk * 16
                    ee = ee_b[pl.ds(off, 16)]
                    for r in range(16):
                        a = ee[r]
                        row = off + r
                        for k in range(H // 16):
                            srows_b[row, pl.ds(k * 16, 16)] = (
                                rows_b[row, pl.ds(k * 16, 16)] * a)
                    return c2

                lax.fori_loop(0, B // 16, group_b, 0)
                pltpu.async_copy(srows_b, acc.at[fidx], ssem_b, add=True)
                pltpu.async_copy(ee_b, wacc.at[fidx], ssem_b, add=True)

        for b in range(2):
            (sidx_b, didx_b, rows_b, srows_b, ldst_b, ee_b, gsem_b,
             ssem_b) = bufs[b]
            fidx = plsc.Indices(ldst_b, ignored_value=-1)
            pltpu.make_async_copy(srows_b, acc.at[fidx], ssem_b).wait()
            pltpu.make_async_copy(ee_b, wacc.at[fidx], ssem_b).wait()

        plsc.subcore_barrier()
        pltpu.sync_copy(acc.at[pl.ds(base, RPQ)],
                        acc_o.at[pl.ds(lo + base, RPQ)])
        pltpu.sync_copy(wacc.at[pl.ds(base, RPQ)], zv)
        pltpu.sync_copy(zv, wacc_o.at[pl.ds(lo + base, RPQ)])
        plsc.subcore_barrier()


def _make_sc_edges():
    mesh = plsc.VectorSubcoreMesh(core_axis_name="c", subcore_axis_name="s")
    kfn = functools.partial(
        pl.kernel,
        out_type=[
            jax.ShapeDtypeStruct((NPAD, H), jnp.float32),
            jax.ShapeDtypeStruct((NPAD,), jnp.float32),
        ],
        mesh=mesh,
        compiler_params=pltpu.CompilerParams(needs_layout_passes=False,
                                             use_tc_tiling_on_sc=False),
        scratch_types=[
            pltpu.VMEM((B,), jnp.int32),
            pltpu.VMEM((B,), jnp.int32),
            pltpu.VMEM((B,), jnp.int32),
            pltpu.VMEM((B,), jnp.int32),
            pltpu.VMEM((B,), jnp.int32),
            pltpu.VMEM((B,), jnp.int32),
            pltpu.VMEM((N,), jnp.float32),
            pltpu.VMEM((N,), jnp.float32),
            pltpu.VMEM((16,), jnp.float32),
            pltpu.VMEM((B, H), jnp.float32),
            pltpu.VMEM((B, H), jnp.float32),
            pltpu.VMEM((B, H), jnp.float32),
            pltpu.VMEM((B, H), jnp.float32),
            pltpu.VMEM((B,), jnp.float32),
            pltpu.VMEM((B,), jnp.float32),
            pltpu.VMEM((RPQ,), jnp.float32),
            pltpu.VMEM_SHARED((ACCR, H), jnp.float32),
            pltpu.VMEM_SHARED((ACCR,), jnp.float32),
            pltpu.SemaphoreType.DMA,
            pltpu.SemaphoreType.DMA,
            pltpu.SemaphoreType.DMA,
            pltpu.SemaphoreType.DMA,
        ],
    )(_sc_body)
    return kfn


_sc_edges = _make_sc_edges()


def _tc2a_body(a_ref, w_ref, bc_ref, y_ref, st_ref, s1_ref, s2_ref):
    i = pl.program_id(0)
    y = a_ref[...] / (w_ref[...] + 1e-16) + bc_ref[...]
    y_ref[...] = y

    @pl.when(i == 0)
    def _():
        s1_ref[...] = jnp.zeros((8, H), jnp.float32)
        s2_ref[...] = jnp.zeros((8, H), jnp.float32)

    s1_ref[0:1, :] = s1_ref[0:1, :] + jnp.sum(y, axis=0, keepdims=True)
    s2_ref[0:1, :] = s2_ref[0:1, :] + jnp.sum(y * y, axis=0, keepdims=True)

    @pl.when(i == ROWS_N - 1)
    def _():
        st_ref[...] = jnp.concatenate([s1_ref[0:1, :], s2_ref[0:1, :]], axis=0)


def _tc2a(accs, waccs, bc):
    return pl.pallas_call(
        _tc2a_body,
        grid=(ROWS_N,),
        in_specs=[
            pl.BlockSpec((BR, H), lambda i: (i, 0)),
            pl.BlockSpec((BR, 1), lambda i: (i, 0)),
            pl.BlockSpec((1, H), lambda i: (0, 0)),
        ],
        out_specs=[
            pl.BlockSpec((BR, H), lambda i: (i, 0)),
            pl.BlockSpec((2, H), lambda i: (0, 0)),
        ],
        out_shape=[
            jax.ShapeDtypeStruct((N, H), jnp.float32),
            jax.ShapeDtypeStruct((2, H), jnp.float32),
        ],
        scratch_shapes=[
            pltpu.VMEM((8, H), jnp.float32),
            pltpu.VMEM((8, H), jnp.float32),
        ],
    )(accs, waccs, bc)


def _tc2b_body(y_ref, st_ref, res_ref, g_ref, be_ref, w2_ref, asrc_ref,
               adst_ref, h1_ref, z_ref, zs_ref, zd_ref, gub_ref, mx_ref):
    i = pl.program_id(0)
    mu = st_ref[0:1, :] * (1.0 / N)
    var = st_ref[1:2, :] * (1.0 / N) - mu * mu
    rs = lax.rsqrt(var + EPS_BN)
    y = y_ref[...]
    h1 = jnp.maximum(g_ref[...] * (y - mu) * rs + be_ref[...], 0.0) + res_ref[...]
    z = jnp.dot(h1, w2_ref[...], preferred_element_type=jnp.float32)
    zs = jnp.sum(z * asrc_ref[...], axis=1, keepdims=True)
    zd = jnp.sum(z * adst_ref[...], axis=1, keepdims=True)
    h1_ref[...] = h1
    z_ref[...] = z
    zs_ref[...] = zs
    zd_ref[...] = zd

    @pl.when(i == 0)
    def _():
        mx_ref[0] = -3e38
        mx_ref[1] = -3e38

    mx_ref[0] = jnp.maximum(mx_ref[0], jnp.max(zs))
    mx_ref[1] = jnp.maximum(mx_ref[1], jnp.max(zd))

    @pl.when(i == ROWS_N - 1)
    def _():
        gub_ref[...] = jnp.full((1, H), mx_ref[0] + mx_ref[1], jnp.float32)


def _tc2b(y, st, res, g, be, w2, asrc, adst):
    full = lambda s: pl.BlockSpec(s, lambda i: (0,) * len(s))
    return pl.pallas_call(
        _tc2b_body,
        grid=(ROWS_N,),
        in_specs=[
            pl.BlockSpec((BR, H), lambda i: (i, 0)),
            full((2, H)),
            pl.BlockSpec((BR, H), lambda i: (i, 0)),
            full((1, H)), full((1, H)), full((H, H)), full((1, H)), full((1, H)),
        ],
        out_specs=[
            pl.BlockSpec((BR, H), lambda i: (i, 0)),
            pl.BlockSpec((BR, H), lambda i: (i, 0)),
            pl.BlockSpec((BR, 1), lambda i: (i, 0)),
            pl.BlockSpec((BR, 1), lambda i: (i, 0)),
            pl.BlockSpec((1, H), lambda i: (0, 0)),
        ],
        out_shape=[
            jax.ShapeDtypeStruct((N, H), jnp.float32),
            jax.ShapeDtypeStruct((N, H), jnp.float32),
            jax.ShapeDtypeStruct((N, 1), jnp.float32),
            jax.ShapeDtypeStruct((N, 1), jnp.float32),
            jax.ShapeDtypeStruct((1, H), jnp.float32),
        ],
        scratch_shapes=[pltpu.SMEM((2,), jnp.float32)],
    )(y, st, res, g, be, w2, asrc, adst)


def _tc3_body(y_ref, st_ref, res_ref, g_ref, be_ref, wf1_ref, bf1_ref,
              wf2_ref, bf2_ref, out_ref):
    mu = st_ref[0:1, :] * (1.0 / N)
    var = st_ref[1:2, :] * (1.0 / N) - mu * mu
    rs = lax.rsqrt(var + EPS_BN)
    y = y_ref[...]
    h2 = jnp.maximum(g_ref[...] * (y - mu) * rs + be_ref[...], 0.0) + res_ref[...]
    f = jnp.maximum(
        jnp.dot(h2, wf1_ref[...], preferred_element_type=jnp.float32)
        + bf1_ref[...], 0.0)
    out_ref[...] = (jnp.dot(f, wf2_ref[...], preferred_element_type=jnp.float32)
                    + bf2_ref[...])


def _tc3(y, st, res, g, be, wf1, bf1, wf2, bf2):
    full = lambda s: pl.BlockSpec(s, lambda i: (0,) * len(s))
    return pl.pallas_call(
        _tc3_body,
        grid=(ROWS_N,),
        in_specs=[
            pl.BlockSpec((BR, H), lambda i: (i, 0)),
            full((2, H)),
            pl.BlockSpec((BR, H), lambda i: (i, 0)),
            full((1, H)), full((1, H)), full((H, H)), full((1, H)),
            full((H, OUT)), full((1, OUT)),
        ],
        out_specs=pl.BlockSpec((BR, OUT), lambda i: (i, 0)),
        out_shape=jax.ShapeDtypeStruct((N, OUT), jnp.float32),
    )(y, st, res, g, be, wf1, bf1, wf2, bf2)


def kernel(x, W_sat, b_sat, W_nei, b_nei, W_fus, b_fus, W1, a_src1, a_dst1,
           bc1, g1, be1, W2, a_src2, a_dst2, bc2, g2, be2, Wf1, bf1, Wf2, bf2,
           edge_index):
    row = lambda v: v.reshape(1, -1)
    si = jnp.arange(N, dtype=edge_index.dtype)
    pad = jnp.zeros((EPAD - E2,), edge_index.dtype)
    srcb = jnp.concatenate([edge_index[0], si, pad]).reshape(NS, NB, B)
    dstb = jnp.concatenate([edge_index[1], si, pad]).reshape(NS, NB, B)

    h0, z1, zs1, zd1, gub1 = _tc1(
        x, W_sat, row(b_sat), W_nei, row(b_nei),
        row(W_fus[:H, 0]), row(W_fus[H:, 0]), b_fus.reshape(1, 1),
        W1, row(a_src1), row(a_dst1))
    acc1, wacc1 = _sc_edges(z1, zs1.reshape(N), zd1.reshape(N),
                            gub1.reshape(H), srcb, dstb)
    y1, st1 = _tc2a(acc1[:N], wacc1[:N].reshape(N, 1), row(bc1))
    h1, z2, zs2, zd2, gub2 = _tc2b(y1, st1, h0, row(g1), row(be1),
                                   W2, row(a_src2), row(a_dst2))
    acc2, wacc2 = _sc_edges(z2, zs2.reshape(N), zd2.reshape(N),
                            gub2.reshape(H), srcb, dstb)
    y2, st2 = _tc2a(acc2[:N], wacc2[:N].reshape(N, 1), row(bc2))
    return _tc3(y2, st2, h1, row(g2), row(be2), Wf1, row(bf1), Wf2, row(bf2))


# speedup vs baseline: 4.4519x; 1.0130x over previous
import functools

import jax
import jax.numpy as jnp
from jax import lax
from jax.experimental import pallas as pl
from jax.experimental.pallas import tpu as pltpu
from jax.experimental.pallas import tpu_sc as plsc

N = 10000
E = 320000
D_IN = 128
SAT = 64
H = 128
OUT = 64
NEG_SLOPE = 0.2
EPS_BN = 1e-5

E2 = E + N
NC = 2
NS = 16
B = 128
NB = 168
CE = NB * B
EPAD = NS * CE
NPAD = 10240
QR = NPAD // 4
ACCR = QR
RPQ = QR // NS

ROWS_N = 10
BR = N // ROWS_N


def _tc1_body(x_ref, wsat_ref, bsat_ref, wnei_ref, bnei_ref, wfa_ref, wfb_ref,
              bfus_ref, w1_ref, asrc_ref, adst_ref,
              h0_ref, z_ref, zs_ref, zd_ref, gub_ref, mx_ref):
    i = pl.program_id(0)
    xb = x_ref[...]
    sat = jnp.maximum(
        jnp.dot(xb[:, :SAT], wsat_ref[...], preferred_element_type=jnp.float32)
        + bsat_ref[...], 0.0)
    nei = jnp.maximum(
        jnp.dot(xb[:, SAT:], wnei_ref[...], preferred_element_type=jnp.float32)
        + bnei_ref[...], 0.0)
    gl = (jnp.sum(sat * wfa_ref[...], axis=1, keepdims=True)
          + jnp.sum(nei * wfb_ref[...], axis=1, keepdims=True)
          + bfus_ref[0, 0])
    gate = jax.nn.sigmoid(gl)
    h = gate * sat + (1.0 - gate) * nei
    z = jnp.dot(h, w1_ref[...], preferred_element_type=jnp.float32)
    zs = jnp.sum(z * asrc_ref[...], axis=1, keepdims=True)
    zd = jnp.sum(z * adst_ref[...], axis=1, keepdims=True)
    h0_ref[...] = h
    z_ref[...] = z
    zs_ref[...] = zs
    zd_ref[...] = zd

    @pl.when(i == 0)
    def _():
        mx_ref[0] = -3e38
        mx_ref[1] = -3e38

    mx_ref[0] = jnp.maximum(mx_ref[0], jnp.max(zs))
    mx_ref[1] = jnp.maximum(mx_ref[1], jnp.max(zd))

    @pl.when(i == ROWS_N - 1)
    def _():
        gub_ref[...] = jnp.full((1, H), mx_ref[0] + mx_ref[1], jnp.float32)


def _tc1(x, wsat, bsat, wnei, bnei, wfa, wfb, bfus, w1, asrc, adst):
    full = lambda s: pl.BlockSpec(s, lambda i: (0,) * len(s))
    return pl.pallas_call(
        _tc1_body,
        grid=(ROWS_N,),
        in_specs=[
            pl.BlockSpec((BR, D_IN), lambda i: (i, 0)),
            full((SAT, H)), full((1, H)), full((D_IN - SAT, H)), full((1, H)),
            full((1, H)), full((1, H)), full((1, 1)), full((H, H)),
            full((1, H)), full((1, H)),
        ],
        out_specs=[
            pl.BlockSpec((BR, H), lambda i: (i, 0)),
            pl.BlockSpec((BR, H), lambda i: (i, 0)),
            pl.BlockSpec((BR, 1), lambda i: (i, 0)),
            pl.BlockSpec((BR, 1), lambda i: (i, 0)),
            pl.BlockSpec((1, H), lambda i: (0, 0)),
        ],
        out_shape=[
            jax.ShapeDtypeStruct((N, H), jnp.float32),
            jax.ShapeDtypeStruct((N, H), jnp.float32),
            jax.ShapeDtypeStruct((N, 1), jnp.float32),
            jax.ShapeDtypeStruct((N, 1), jnp.float32),
            jax.ShapeDtypeStruct((1, H), jnp.float32),
        ],
        scratch_shapes=[pltpu.SMEM((2,), jnp.float32)],
    )(x, wsat, bsat, wnei, bnei, wfa, wfb, bfus, w1, asrc, adst)


def _sc_body(z_h, zs_h, zd_h, g_h, ed_h, acc_o, wacc_o,
             exb0, exb1, ldst0, ldst1, zs_v, zd_v, g_v,
             rows0, rows1, srows0, srows1, ee0, ee1, zv, acc, wacc,
             gsem0, gsem1, ssem0, ssem1):
    cid = lax.axis_index("c")
    sid = lax.axis_index("s")

    pltpu.sync_copy(zs_h, zs_v)
    pltpu.sync_copy(zd_h, zd_v)
    pltpu.sync_copy(g_h.at[pl.ds(0, 16)], g_v)

    g16 = g_v[...]
    valid = E2 - sid * CE
    lanes = lax.iota(jnp.int32, 16)
    zero16 = jnp.zeros((16,), jnp.float32)
    base = sid * RPQ
    bufs = ((exb0, rows0, srows0, ldst0, ee0, gsem0, ssem0),
            (exb1, rows1, srows1, ldst1, ee1, gsem1, ssem1))

    for p in range(2):
        q = 2 * p + cid
        lo = q * QR

        def zrow(r, c):
            for k in range(H // 16):
                srows0[r, pl.ds(k * 16, 16)] = zero16
            return c

        lax.fori_loop(0, B, zrow, 0)

        def zw(r, c):
            zv[pl.ds(r * 16, 16)] = zero16
            return c

        lax.fori_loop(0, RPQ // 16, zw, 0)
        pltpu.sync_copy(srows0, acc.at[pl.ds(base, B)])
        pltpu.sync_copy(srows0.at[pl.ds(0, RPQ - B)],
                        acc.at[pl.ds(base + B, RPQ - B)])
        pltpu.sync_copy(zv, wacc.at[pl.ds(base, RPQ)])
        plsc.subcore_barrier()

        pltpu.sync_copy(ed_h.at[sid, pl.ds(0, 2)], exb0)
        pltpu.async_copy(z_h.at[exb0.at[0]], rows0, gsem0)

        @pl.loop(0, NB // 2)
        def pair(jj):
            for b in range(2):
                exb_b, rows_b, srows_b, ldst_b, ee_b, gsem_b, ssem_b = bufs[b]
                exb_n, rows_n = bufs[1 - b][0], bufs[1 - b][1]
                gsem_n = bufs[1 - b][5]
                j = 2 * jj + b
                fidx = plsc.Indices(ldst_b, ignored_value=-1)

                @pl.when(j + 1 < NB)
                def _():
                    pltpu.sync_copy(ed_h.at[sid, pl.ds(2 * (j + 1), 2)], exb_n)
                    pltpu.async_copy(z_h.at[exb_n.at[0]], rows_n, gsem_n)

                @pl.when(j >= 2)
                def _():
                    pltpu.make_async_copy(
                        srows_b, acc.at[fidx], ssem_b).wait()
                    pltpu.make_async_copy(
                        ee_b, wacc.at[fidx], ssem_b).wait()

                def group_a(kk, c2):
                    off = kk * 16
                    s16 = exb_b[0, pl.ds(off, 16)]
                    d16 = exb_b[1, pl.ds(off, 16)]
                    zsg = plsc.load_gather(zs_v, [s16])
                    zdg = plsc.load_gather(zd_v, [d16])
                    e = zsg + zdg
                    e = jnp.where(e >= 0.0, e, e * NEG_SLOPE)
                    ee = jnp.exp(e - g16)
                    ee = jnp.where((j * B + off + lanes) < valid, ee, 0.0)
                    ld = d16 - lo
                    ld = jnp.where((ld >= 0) & (ld < QR), ld, -1)
                    ldst_b[pl.ds(off, 16)] = ld
                    ee_b[pl.ds(off, 16)] = ee
                    return c2

                lax.fori_loop(0, B // 16, group_a, 0)
                pltpu.make_async_copy(z_h.at[exb_b.at[0]], rows_b,
                                      gsem_b).wait()

                def group_b(kk, c2):
                    off = kk * 16
                    for r in range(16):
                        row = off + r
                        av = plsc.load_gather(
                            ee_b, [jnp.full((16,), row, jnp.int32)])
                        for k in range(H // 16):
                            srows_b[row, pl.ds(k * 16, 16)] = (
                                rows_b[row, pl.ds(k * 16, 16)] * av)
                    return c2

                lax.fori_loop(0, B // 16, group_b, 0)
                pltpu.async_copy(srows_b, acc.at[fidx], ssem_b, add=True)
                pltpu.async_copy(ee_b, wacc.at[fidx], ssem_b, add=True)

        for b in range(2):
            exb_b, rows_b, srows_b, ldst_b, ee_b, gsem_b, ssem_b = bufs[b]
            fidx = plsc.Indices(ldst_b, ignored_value=-1)
            pltpu.make_async_copy(srows_b, acc.at[fidx], ssem_b).wait()
            pltpu.make_async_copy(ee_b, wacc.at[fidx], ssem_b).wait()

        plsc.subcore_barrier()
        pltpu.sync_copy(acc.at[pl.ds(base, RPQ)],
                        acc_o.at[pl.ds(lo + base, RPQ)])
        pltpu.sync_copy(wacc.at[pl.ds(base, RPQ)], zv)
        pltpu.sync_copy(zv, wacc_o.at[pl.ds(lo + base, RPQ)])
        plsc.subcore_barrier()


def _make_sc_edges():
    mesh = plsc.VectorSubcoreMesh(core_axis_name="c", subcore_axis_name="s")
    kfn = functools.partial(
        pl.kernel,
        out_type=[
            jax.ShapeDtypeStruct((NPAD, H), jnp.float32),
            jax.ShapeDtypeStruct((NPAD,), jnp.float32),
        ],
        mesh=mesh,
        compiler_params=pltpu.CompilerParams(needs_layout_passes=False,
                                             use_tc_tiling_on_sc=False),
        scratch_types=[
            pltpu.VMEM((2, B), jnp.int32),
            pltpu.VMEM((2, B), jnp.int32),
            pltpu.VMEM((B,), jnp.int32),
            pltpu.VMEM((B,), jnp.int32),
            pltpu.VMEM((N,), jnp.float32),
            pltpu.VMEM((N,), jnp.float32),
            pltpu.VMEM((16,), jnp.float32),
            pltpu.VMEM((B, H), jnp.float32),
            pltpu.VMEM((B, H), jnp.float32),
            pltpu.VMEM((B, H), jnp.float32),
            pltpu.VMEM((B, H), jnp.float32),
            pltpu.VMEM((B,), jnp.float32),
            pltpu.VMEM((B,), jnp.float32),
            pltpu.VMEM((RPQ,), jnp.float32),
            pltpu.VMEM_SHARED((ACCR, H), jnp.float32),
            pltpu.VMEM_SHARED((ACCR,), jnp.float32),
            pltpu.SemaphoreType.DMA,
            pltpu.SemaphoreType.DMA,
            pltpu.SemaphoreType.DMA,
            pltpu.SemaphoreType.DMA,
        ],
    )(_sc_body)
    return kfn


_sc_edges = _make_sc_edges()


def _tc2a_body(a_ref, w_ref, bc_ref, y_ref, st_ref, s1_ref, s2_ref):
    i = pl.program_id(0)
    y = a_ref[...] / (w_ref[...] + 1e-16) + bc_ref[...]
    y_ref[...] = y

    @pl.when(i == 0)
    def _():
        s1_ref[...] = jnp.zeros((8, H), jnp.float32)
        s2_ref[...] = jnp.zeros((8, H), jnp.float32)

    s1_ref[0:1, :] = s1_ref[0:1, :] + jnp.sum(y, axis=0, keepdims=True)
    s2_ref[0:1, :] = s2_ref[0:1, :] + jnp.sum(y * y, axis=0, keepdims=True)

    @pl.when(i == ROWS_N - 1)
    def _():
        st_ref[...] = jnp.concatenate([s1_ref[0:1, :], s2_ref[0:1, :]], axis=0)


def _tc2a(accs, waccs, bc):
    return pl.pallas_call(
        _tc2a_body,
        grid=(ROWS_N,),
        in_specs=[
            pl.BlockSpec((BR, H), lambda i: (i, 0)),
            pl.BlockSpec((BR, 1), lambda i: (i, 0)),
            pl.BlockSpec((1, H), lambda i: (0, 0)),
        ],
        out_specs=[
            pl.BlockSpec((BR, H), lambda i: (i, 0)),
            pl.BlockSpec((2, H), lambda i: (0, 0)),
        ],
        out_shape=[
            jax.ShapeDtypeStruct((N, H), jnp.float32),
            jax.ShapeDtypeStruct((2, H), jnp.float32),
        ],
        scratch_shapes=[
            pltpu.VMEM((8, H), jnp.float32),
            pltpu.VMEM((8, H), jnp.float32),
        ],
    )(accs, waccs, bc)


def _tc2b_body(y_ref, st_ref, res_ref, g_ref, be_ref, w2_ref, asrc_ref,
               adst_ref, h1_ref, z_ref, zs_ref, zd_ref, gub_ref, mx_ref):
    i = pl.program_id(0)
    mu = st_ref[0:1, :] * (1.0 / N)
    var = st_ref[1:2, :] * (1.0 / N) - mu * mu
    rs = lax.rsqrt(var + EPS_BN)
    y = y_ref[...]
    h1 = jnp.maximum(g_ref[...] * (y - mu) * rs + be_ref[...], 0.0) + res_ref[...]
    z = jnp.dot(h1, w2_ref[...], preferred_element_type=jnp.float32)
    zs = jnp.sum(z * asrc_ref[...], axis=1, keepdims=True)
    zd = jnp.sum(z * adst_ref[...], axis=1, keepdims=True)
    h1_ref[...] = h1
    z_ref[...] = z
    zs_ref[...] = zs
    zd_ref[...] = zd

    @pl.when(i == 0)
    def _():
        mx_ref[0] = -3e38
        mx_ref[1] = -3e38

    mx_ref[0] = jnp.maximum(mx_ref[0], jnp.max(zs))
    mx_ref[1] = jnp.maximum(mx_ref[1], jnp.max(zd))

    @pl.when(i == ROWS_N - 1)
    def _():
        gub_ref[...] = jnp.full((1, H), mx_ref[0] + mx_ref[1], jnp.float32)


def _tc2b(y, st, res, g, be, w2, asrc, adst):
    full = lambda s: pl.BlockSpec(s, lambda i: (0,) * len(s))
    return pl.pallas_call(
        _tc2b_body,
        grid=(ROWS_N,),
        in_specs=[
            pl.BlockSpec((BR, H), lambda i: (i, 0)),
            full((2, H)),
            pl.BlockSpec((BR, H), lambda i: (i, 0)),
            full((1, H)), full((1, H)), full((H, H)), full((1, H)), full((1, H)),
        ],
        out_specs=[
            pl.BlockSpec((BR, H), lambda i: (i, 0)),
            pl.BlockSpec((BR, H), lambda i: (i, 0)),
            pl.BlockSpec((BR, 1), lambda i: (i, 0)),
            pl.BlockSpec((BR, 1), lambda i: (i, 0)),
            pl.BlockSpec((1, H), lambda i: (0, 0)),
        ],
        out_shape=[
            jax.ShapeDtypeStruct((N, H), jnp.float32),
            jax.ShapeDtypeStruct((N, H), jnp.float32),
            jax.ShapeDtypeStruct((N, 1), jnp.float32),
            jax.ShapeDtypeStruct((N, 1), jnp.float32),
            jax.ShapeDtypeStruct((1, H), jnp.float32),
        ],
        scratch_shapes=[pltpu.SMEM((2,), jnp.float32)],
    )(y, st, res, g, be, w2, asrc, adst)


def _tc3_body(y_ref, st_ref, res_ref, g_ref, be_ref, wf1_ref, bf1_ref,
              wf2_ref, bf2_ref, out_ref):
    mu = st_ref[0:1, :] * (1.0 / N)
    var = st_ref[1:2, :] * (1.0 / N) - mu * mu
    rs = lax.rsqrt(var + EPS_BN)
    y = y_ref[...]
    h2 = jnp.maximum(g_ref[...] * (y - mu) * rs + be_ref[...], 0.0) + res_ref[...]
    f = jnp.maximum(
        jnp.dot(h2, wf1_ref[...], preferred_element_type=jnp.float32)
        + bf1_ref[...], 0.0)
    out_ref[...] = (jnp.dot(f, wf2_ref[...], preferred_element_type=jnp.float32)
                    + bf2_ref[...])


def _tc3(y, st, res, g, be, wf1, bf1, wf2, bf2):
    full = lambda s: pl.BlockSpec(s, lambda i: (0,) * len(s))
    return pl.pallas_call(
        _tc3_body,
        grid=(ROWS_N,),
        in_specs=[
            pl.BlockSpec((BR, H), lambda i: (i, 0)),
            full((2, H)),
            pl.BlockSpec((BR, H), lambda i: (i, 0)),
            full((1, H)), full((1, H)), full((H, H)), full((1, H)),
            full((H, OUT)), full((1, OUT)),
        ],
        out_specs=pl.BlockSpec((BR, OUT), lambda i: (i, 0)),
        out_shape=jax.ShapeDtypeStruct((N, OUT), jnp.float32),
    )(y, st, res, g, be, wf1, bf1, wf2, bf2)


def kernel(x, W_sat, b_sat, W_nei, b_nei, W_fus, b_fus, W1, a_src1, a_dst1,
           bc1, g1, be1, W2, a_src2, a_dst2, bc2, g2, be2, Wf1, bf1, Wf2, bf2,
           edge_index):
    row = lambda v: v.reshape(1, -1)
    si = jnp.arange(N, dtype=edge_index.dtype)
    pad = jnp.zeros((EPAD - E2,), edge_index.dtype)
    srcb = jnp.concatenate([edge_index[0], si, pad]).reshape(NS, NB, 1, B)
    dstb = jnp.concatenate([edge_index[1], si, pad]).reshape(NS, NB, 1, B)
    edb = jnp.concatenate([srcb, dstb], axis=2).reshape(NS, NB * 2, B)

    h0, z1, zs1, zd1, gub1 = _tc1(
        x, W_sat, row(b_sat), W_nei, row(b_nei),
        row(W_fus[:H, 0]), row(W_fus[H:, 0]), b_fus.reshape(1, 1),
        W1, row(a_src1), row(a_dst1))
    acc1, wacc1 = _sc_edges(z1, zs1.reshape(N), zd1.reshape(N),
                            gub1.reshape(H), edb)
    y1, st1 = _tc2a(acc1[:N], wacc1[:N].reshape(N, 1), row(bc1))
    h1, z2, zs2, zd2, gub2 = _tc2b(y1, st1, h0, row(g1), row(be1),
                                   W2, row(a_src2), row(a_dst2))
    acc2, wacc2 = _sc_edges(z2, zs2.reshape(N), zd2.reshape(N),
                            gub2.reshape(H), edb)
    y2, st2 = _tc2a(acc2[:N], wacc2[:N].reshape(N, 1), row(bc2))
    return _tc3(y2, st2, h1, row(g2), row(be2), Wf1, row(bf1), Wf2, row(bf2))


# speedup vs baseline: 6.8849x; 1.5465x over previous
import functools

import jax
import jax.numpy as jnp
from jax import lax
from jax.experimental import pallas as pl
from jax.experimental.pallas import tpu as pltpu
from jax.experimental.pallas import tpu_sc as plsc

N = 10000
E = 320000
D_IN = 128
SAT = 64
H = 128
OUT = 64
NEG_SLOPE = 0.2
EPS_BN = 1e-5

E2 = E + N
NC = 2
NS = 16
B = 128
NB = 168
CE = NB * B
EPAD = NS * CE
NPAD = 10240
QR = NPAD // 4
ACCR = QR
RPQ = QR // NS

ROWS_N = 10
BR = N // ROWS_N


def _tc1_body(x_ref, wsat_ref, bsat_ref, wnei_ref, bnei_ref, wfa_ref, wfb_ref,
              bfus_ref, w1_ref, asrc_ref, adst_ref,
              h0_ref, z_ref, zb_ref, zs_ref, zd_ref, gub_ref, mx_ref):
    i = pl.program_id(0)
    xb = x_ref[...]
    sat = jnp.maximum(
        jnp.dot(xb[:, :SAT], wsat_ref[...], preferred_element_type=jnp.float32)
        + bsat_ref[...], 0.0)
    nei = jnp.maximum(
        jnp.dot(xb[:, SAT:], wnei_ref[...], preferred_element_type=jnp.float32)
        + bnei_ref[...], 0.0)
    gl = (jnp.sum(sat * wfa_ref[...], axis=1, keepdims=True)
          + jnp.sum(nei * wfb_ref[...], axis=1, keepdims=True)
          + bfus_ref[0, 0])
    gate = jax.nn.sigmoid(gl)
    h = gate * sat + (1.0 - gate) * nei
    z = jnp.dot(h, w1_ref[...], preferred_element_type=jnp.float32)
    zs = jnp.sum(z * asrc_ref[...], axis=1, keepdims=True)
    zd = jnp.sum(z * adst_ref[...], axis=1, keepdims=True)
    h0_ref[...] = h
    z_ref[...] = z
    zb_ref[...] = z.astype(jnp.bfloat16)
    zs_ref[...] = zs
    zd_ref[...] = zd

    @pl.when(i == 0)
    def _():
        mx_ref[0] = -3e38
        mx_ref[1] = -3e38

    mx_ref[0] = jnp.maximum(mx_ref[0], jnp.max(zs))
    mx_ref[1] = jnp.maximum(mx_ref[1], jnp.max(zd))

    @pl.when(i == ROWS_N - 1)
    def _():
        gub_ref[...] = jnp.full((1, H), mx_ref[0] + mx_ref[1], jnp.float32)


def _tc1(x, wsat, bsat, wnei, bnei, wfa, wfb, bfus, w1, asrc, adst):
    full = lambda s: pl.BlockSpec(s, lambda i: (0,) * len(s))
    return pl.pallas_call(
        _tc1_body,
        grid=(ROWS_N,),
        in_specs=[
            pl.BlockSpec((BR, D_IN), lambda i: (i, 0)),
            full((SAT, H)), full((1, H)), full((D_IN - SAT, H)), full((1, H)),
            full((1, H)), full((1, H)), full((1, 1)), full((H, H)),
            full((1, H)), full((1, H)),
        ],
        out_specs=[
            pl.BlockSpec((BR, H), lambda i: (i, 0)),
            pl.BlockSpec((BR, H), lambda i: (i, 0)),
            pl.BlockSpec((BR, H), lambda i: (i, 0)),
            pl.BlockSpec((BR, 1), lambda i: (i, 0)),
            pl.BlockSpec((BR, 1), lambda i: (i, 0)),
            pl.BlockSpec((1, H), lambda i: (0, 0)),
        ],
        out_shape=[
            jax.ShapeDtypeStruct((N, H), jnp.float32),
            jax.ShapeDtypeStruct((N, H), jnp.float32),
            jax.ShapeDtypeStruct((N, H), jnp.bfloat16),
            jax.ShapeDtypeStruct((N, 1), jnp.float32),
            jax.ShapeDtypeStruct((N, 1), jnp.float32),
            jax.ShapeDtypeStruct((1, H), jnp.float32),
        ],
        scratch_shapes=[pltpu.SMEM((2,), jnp.float32)],
    )(x, wsat, bsat, wnei, bnei, wfa, wfb, bfus, w1, asrc, adst)


def _sc_body(z_h, zs_h, zd_h, g_h, ed_h, acc_o, wacc_o,
             exb0, exb1, ldst0, ldst1, zs_v, zd_v, g_v,
             rows0, rows1, srows0, srows1, ee0, ee1, zv, acc, wacc,
             gsem0, gsem1, ssem0, ssem1):
    cid = lax.axis_index("c")
    sid = lax.axis_index("s")

    pltpu.sync_copy(zs_h, zs_v)
    pltpu.sync_copy(zd_h, zd_v)
    pltpu.sync_copy(g_h.at[pl.ds(0, 16)], g_v)

    g16 = g_v[...]
    valid = E2 - sid * CE
    lanes = lax.iota(jnp.int32, 16)
    zero16 = jnp.zeros((16,), jnp.float32)
    base = sid * RPQ
    bufs = ((exb0, rows0, srows0, ldst0, ee0, gsem0, ssem0),
            (exb1, rows1, srows1, ldst1, ee1, gsem1, ssem1))

    for p in range(2):
        q = 2 * p + cid
        lo = q * QR

        def zrow(r, c):
            for k in range(H // 16):
                srows0[r, pl.ds(k * 16, 16)] = zero16
            return c

        lax.fori_loop(0, B, zrow, 0)

        def zw(r, c):
            zv[pl.ds(r * 16, 16)] = zero16
            return c

        lax.fori_loop(0, RPQ // 16, zw, 0)
        pltpu.sync_copy(srows0, acc.at[pl.ds(base, B)])
        pltpu.sync_copy(srows0.at[pl.ds(0, RPQ - B)],
                        acc.at[pl.ds(base + B, RPQ - B)])
        pltpu.sync_copy(zv, wacc.at[pl.ds(base, RPQ)])
        plsc.subcore_barrier()

        pltpu.sync_copy(ed_h.at[sid, pl.ds(0, 2)], exb0)
        pltpu.async_copy(z_h.at[exb0.at[0]], rows0, gsem0)

        @pl.loop(0, NB // 2)
        def pair(jj):
            for b in range(2):
                exb_b, rows_b, srows_b, ldst_b, ee_b, gsem_b, ssem_b = bufs[b]
                exb_n, rows_n = bufs[1 - b][0], bufs[1 - b][1]
                gsem_n = bufs[1 - b][5]
                j = 2 * jj + b
                fidx = plsc.Indices(ldst_b, ignored_value=-1)

                @pl.when(j + 1 < NB)
                def _():
                    pltpu.sync_copy(ed_h.at[sid, pl.ds(2 * (j + 1), 2)], exb_n)
                    pltpu.async_copy(z_h.at[exb_n.at[0]], rows_n, gsem_n)

                @pl.when(j >= 2)
                def _():
                    pltpu.make_async_copy(
                        srows_b, acc.at[fidx], ssem_b).wait()
                    pltpu.make_async_copy(
                        ee_b, wacc.at[fidx], ssem_b).wait()

                def group_a(kk, c2):
                    off = kk * 16
                    s16 = exb_b[0, pl.ds(off, 16)]
                    d16 = exb_b[1, pl.ds(off, 16)]
                    zsg = plsc.load_gather(zs_v, [s16])
                    zdg = plsc.load_gather(zd_v, [d16])
                    e = zsg + zdg
                    e = jnp.where(e >= 0.0, e, e * NEG_SLOPE)
                    ee = jnp.exp(e - g16)
                    ee = jnp.where((j * B + off + lanes) < valid, ee, 0.0)
                    ld = d16 - lo
                    ld = jnp.where((ld >= 0) & (ld < QR), ld, -1)
                    ldst_b[pl.ds(off, 16)] = ld
                    ee_b[pl.ds(off, 16)] = ee
                    return c2

                lax.fori_loop(0, B // 16, group_a, 0)
                pltpu.make_async_copy(z_h.at[exb_b.at[0]], rows_b,
                                      gsem_b).wait()

                def group_b(kk, c2):
                    off = kk * 16
                    for r in range(16):
                        row = off + r
                        av = plsc.load_gather(
                            ee_b, [jnp.full((16,), row, jnp.int32)])
                        for k in range(H // 32):
                            w32 = rows_b[row, pl.ds(k * 32, 32)]
                            lo16, hi16 = plsc.unpack(
                                w32, format=plsc.PackFormat.INTERLEAVED)
                            srows_b[row, pl.ds(k * 32, 16)] = lo16 * av
                            srows_b[row, pl.ds(k * 32 + 16, 16)] = hi16 * av
                    return c2

                lax.fori_loop(0, B // 16, group_b, 0)
                pltpu.async_copy(srows_b, acc.at[fidx], ssem_b, add=True)
                pltpu.async_copy(ee_b, wacc.at[fidx], ssem_b, add=True)

        for b in range(2):
            exb_b, rows_b, srows_b, ldst_b, ee_b, gsem_b, ssem_b = bufs[b]
            fidx = plsc.Indices(ldst_b, ignored_value=-1)
            pltpu.make_async_copy(srows_b, acc.at[fidx], ssem_b).wait()
            pltpu.make_async_copy(ee_b, wacc.at[fidx], ssem_b).wait()

        plsc.subcore_barrier()
        pltpu.sync_copy(acc.at[pl.ds(base, RPQ)],
                        acc_o.at[pl.ds(lo + base, RPQ)])
        pltpu.sync_copy(wacc.at[pl.ds(base, RPQ)], zv)
        pltpu.sync_copy(zv, wacc_o.at[pl.ds(lo + base, RPQ)])
        plsc.subcore_barrier()


def _make_sc_edges():
    mesh = plsc.VectorSubcoreMesh(core_axis_name="c", subcore_axis_name="s")
    kfn = functools.partial(
        pl.kernel,
        out_type=[
            jax.ShapeDtypeStruct((NPAD, H), jnp.float32),
            jax.ShapeDtypeStruct((NPAD,), jnp.float32),
        ],
        mesh=mesh,
        compiler_params=pltpu.CompilerParams(needs_layout_passes=False,
                                             use_tc_tiling_on_sc=False),
        scratch_types=[
            pltpu.VMEM((2, B), jnp.int32),
            pltpu.VMEM((2, B), jnp.int32),
            pltpu.VMEM((B,), jnp.int32),
            pltpu.VMEM((B,), jnp.int32),
            pltpu.VMEM((N,), jnp.float32),
            pltpu.VMEM((N,), jnp.float32),
            pltpu.VMEM((16,), jnp.float32),
            pltpu.VMEM((B, H), jnp.bfloat16),
            pltpu.VMEM((B, H), jnp.bfloat16),
            pltpu.VMEM((B, H), jnp.float32),
            pltpu.VMEM((B, H), jnp.float32),
            pltpu.VMEM((B,), jnp.float32),
            pltpu.VMEM((B,), jnp.float32),
            pltpu.VMEM((RPQ,), jnp.float32),
            pltpu.VMEM_SHARED((ACCR, H), jnp.float32),
            pltpu.VMEM_SHARED((ACCR,), jnp.float32),
            pltpu.SemaphoreType.DMA,
            pltpu.SemaphoreType.DMA,
            pltpu.SemaphoreType.DMA,
            pltpu.SemaphoreType.DMA,
        ],
    )(_sc_body)
    return kfn


_sc_edges = _make_sc_edges()


def _tc2a_body(a_ref, w_ref, bc_ref, y_ref, st_ref, s1_ref, s2_ref):
    i = pl.program_id(0)
    y = a_ref[...] / (w_ref[...] + 1e-16) + bc_ref[...]
    y_ref[...] = y

    @pl.when(i == 0)
    def _():
        s1_ref[...] = jnp.zeros((8, H), jnp.float32)
        s2_ref[...] = jnp.zeros((8, H), jnp.float32)

    s1_ref[0:1, :] = s1_ref[0:1, :] + jnp.sum(y, axis=0, keepdims=True)
    s2_ref[0:1, :] = s2_ref[0:1, :] + jnp.sum(y * y, axis=0, keepdims=True)

    @pl.when(i == ROWS_N - 1)
    def _():
        st_ref[...] = jnp.concatenate([s1_ref[0:1, :], s2_ref[0:1, :]], axis=0)


def _tc2a(accs, waccs, bc):
    return pl.pallas_call(
        _tc2a_body,
        grid=(ROWS_N,),
        in_specs=[
            pl.BlockSpec((BR, H), lambda i: (i, 0)),
            pl.BlockSpec((BR, 1), lambda i: (i, 0)),
            pl.BlockSpec((1, H), lambda i: (0, 0)),
        ],
        out_specs=[
            pl.BlockSpec((BR, H), lambda i: (i, 0)),
            pl.BlockSpec((2, H), lambda i: (0, 0)),
        ],
        out_shape=[
            jax.ShapeDtypeStruct((N, H), jnp.float32),
            jax.ShapeDtypeStruct((2, H), jnp.float32),
        ],
        scratch_shapes=[
            pltpu.VMEM((8, H), jnp.float32),
            pltpu.VMEM((8, H), jnp.float32),
        ],
    )(accs, waccs, bc)


def _tc2b_body(y_ref, st_ref, res_ref, g_ref, be_ref, w2_ref, asrc_ref,
               adst_ref, h1_ref, z_ref, zb_ref, zs_ref, zd_ref, gub_ref,
               mx_ref):
    i = pl.program_id(0)
    mu = st_ref[0:1, :] * (1.0 / N)
    var = st_ref[1:2, :] * (1.0 / N) - mu * mu
    rs = lax.rsqrt(var + EPS_BN)
    y = y_ref[...]
    h1 = jnp.maximum(g_ref[...] * (y - mu) * rs + be_ref[...], 0.0) + res_ref[...]
    z = jnp.dot(h1, w2_ref[...], preferred_element_type=jnp.float32)
    zs = jnp.sum(z * asrc_ref[...], axis=1, keepdims=True)
    zd = jnp.sum(z * adst_ref[...], axis=1, keepdims=True)
    h1_ref[...] = h1
    z_ref[...] = z
    zb_ref[...] = z.astype(jnp.bfloat16)
    zs_ref[...] = zs
    zd_ref[...] = zd

    @pl.when(i == 0)
    def _():
        mx_ref[0] = -3e38
        mx_ref[1] = -3e38

    mx_ref[0] = jnp.maximum(mx_ref[0], jnp.max(zs))
    mx_ref[1] = jnp.maximum(mx_ref[1], jnp.max(zd))

    @pl.when(i == ROWS_N - 1)
    def _():
        gub_ref[...] = jnp.full((1, H), mx_ref[0] + mx_ref[1], jnp.float32)


def _tc2b(y, st, res, g, be, w2, asrc, adst):
    full = lambda s: pl.BlockSpec(s, lambda i: (0,) * len(s))
    return pl.pallas_call(
        _tc2b_body,
        grid=(ROWS_N,),
        in_specs=[
            pl.BlockSpec((BR, H), lambda i: (i, 0)),
            full((2, H)),
            pl.BlockSpec((BR, H), lambda i: (i, 0)),
            full((1, H)), full((1, H)), full((H, H)), full((1, H)), full((1, H)),
        ],
        out_specs=[
            pl.BlockSpec((BR, H), lambda i: (i, 0)),
            pl.BlockSpec((BR, H), lambda i: (i, 0)),
            pl.BlockSpec((BR, H), lambda i: (i, 0)),
            pl.BlockSpec((BR, 1), lambda i: (i, 0)),
            pl.BlockSpec((BR, 1), lambda i: (i, 0)),
            pl.BlockSpec((1, H), lambda i: (0, 0)),
        ],
        out_shape=[
            jax.ShapeDtypeStruct((N, H), jnp.float32),
            jax.ShapeDtypeStruct((N, H), jnp.float32),
            jax.ShapeDtypeStruct((N, H), jnp.bfloat16),
            jax.ShapeDtypeStruct((N, 1), jnp.float32),
            jax.ShapeDtypeStruct((N, 1), jnp.float32),
            jax.ShapeDtypeStruct((1, H), jnp.float32),
        ],
        scratch_shapes=[pltpu.SMEM((2,), jnp.float32)],
    )(y, st, res, g, be, w2, asrc, adst)


def _tc3_body(y_ref, st_ref, res_ref, g_ref, be_ref, wf1_ref, bf1_ref,
              wf2_ref, bf2_ref, out_ref):
    mu = st_ref[0:1, :] * (1.0 / N)
    var = st_ref[1:2, :] * (1.0 / N) - mu * mu
    rs = lax.rsqrt(var + EPS_BN)
    y = y_ref[...]
    h2 = jnp.maximum(g_ref[...] * (y - mu) * rs + be_ref[...], 0.0) + res_ref[...]
    f = jnp.maximum(
        jnp.dot(h2, wf1_ref[...], preferred_element_type=jnp.float32)
        + bf1_ref[...], 0.0)
    out_ref[...] = (jnp.dot(f, wf2_ref[...], preferred_element_type=jnp.float32)
                    + bf2_ref[...])


def _tc3(y, st, res, g, be, wf1, bf1, wf2, bf2):
    full = lambda s: pl.BlockSpec(s, lambda i: (0,) * len(s))
    return pl.pallas_call(
        _tc3_body,
        grid=(ROWS_N,),
        in_specs=[
            pl.BlockSpec((BR, H), lambda i: (i, 0)),
            full((2, H)),
            pl.BlockSpec((BR, H), lambda i: (i, 0)),
            full((1, H)), full((1, H)), full((H, H)), full((1, H)),
            full((H, OUT)), full((1, OUT)),
        ],
        out_specs=pl.BlockSpec((BR, OUT), lambda i: (i, 0)),
        out_shape=jax.ShapeDtypeStruct((N, OUT), jnp.float32),
    )(y, st, res, g, be, wf1, bf1, wf2, bf2)


def kernel(x, W_sat, b_sat, W_nei, b_nei, W_fus, b_fus, W1, a_src1, a_dst1,
           bc1, g1, be1, W2, a_src2, a_dst2, bc2, g2, be2, Wf1, bf1, Wf2, bf2,
           edge_index):
    row = lambda v: v.reshape(1, -1)
    si = jnp.arange(N, dtype=edge_index.dtype)
    pad = jnp.zeros((EPAD - E2,), edge_index.dtype)
    srcb = jnp.concatenate([edge_index[0], si, pad]).reshape(NS, NB, 1, B)
    dstb = jnp.concatenate([edge_index[1], si, pad]).reshape(NS, NB, 1, B)
    edb = jnp.concatenate([srcb, dstb], axis=2).reshape(NS, NB * 2, B)

    import numpy as _np
    _pm = _np.arange(H).reshape(H // 32, 2, 16).transpose(0, 2, 1).reshape(H)
    perm = jnp.asarray(_pm, jnp.int32)
    h0, z1, zb1, zs1, zd1, gub1 = _tc1(
        x, W_sat, row(b_sat), W_nei, row(b_nei),
        row(W_fus[:H, 0]), row(W_fus[H:, 0]), b_fus.reshape(1, 1),
        W1, row(a_src1), row(a_dst1))
    acc1, wacc1 = _sc_edges(zb1[:, perm], zs1.reshape(N), zd1.reshape(N),
                            gub1.reshape(H), edb)
    y1, st1 = _tc2a(acc1[:N], wacc1[:N].reshape(N, 1), row(bc1))
    h1, z2, zb2, zs2, zd2, gub2 = _tc2b(y1, st1, h0, row(g1), row(be1),
                                   W2, row(a_src2), row(a_dst2))
    acc2, wacc2 = _sc_edges(zb2[:, perm], zs2.reshape(N), zd2.reshape(N),
                            gub2.reshape(H), edb)
    y2, st2 = _tc2a(acc2[:N], wacc2[:N].reshape(N, 1), row(bc2))
    return _tc3(y2, st2, h1, row(g2), row(be2), Wf1, row(bf1), Wf2, row(bf2))
